# baseline (speedup 1.0000x reference)
import functools

import jax
import jax.numpy as jnp
from jax import lax
from jax.experimental import pallas as pl
from jax.experimental.pallas import tpu as pltpu
from jax.experimental.pallas import tpu_sc as plsc

N = 10000
E = 320000
R = 2
T = 3
D = 128
RT = R * T

NC = 2
NS = 16
SETS_PER_SC = RT // NC
EPT = E // NS
K = 80
NJC = EPT // K
NJB = 50
NSLAB = NJC // NJB
NBUF = 2
NPAD = 10240
RPT = NPAD // NS
ZR = 64

_PREC = jax.lax.Precision.HIGHEST


def _sc_body(x2, srcv, dstv, zrow, zdeg, agg_out, deg_out,
             src_idx, dst_idx, rows, ones_v, zrow_v, zdeg_v,
             agg_sp, deg_sp, gsem, ssem, dsem, zsem):
  c = lax.axis_index("c")
  s = lax.axis_index("s")
  for i in range(K // 16):
    ones_v[pl.ds(i * 16, 16)] = jnp.ones((16,), jnp.float32)
  pltpu.sync_copy(zrow, zrow_v)
  pltpu.sync_copy(zdeg, zdeg_v)
  for sl in range(SETS_PER_SC):
    sg = c * SETS_PER_SC + sl
    for z in range(RPT // ZR):
      pltpu.async_copy(zrow_v, agg_sp.at[pl.ds(s * RPT + z * ZR, ZR), :],
                       zsem)
    pltpu.sync_copy(zdeg_v, deg_sp.at[pl.ds(s * RPT, RPT)])
    for z in range(RPT // ZR):
      pltpu.make_async_copy(
          zrow_v, agg_sp.at[pl.ds(s * RPT + z * ZR, ZR), :], zsem).wait()
    plsc.subcore_barrier()
    for slab in range(NSLAB):
      pltpu.sync_copy(srcv.at[sg, s, slab], src_idx)
      pltpu.sync_copy(dstv.at[sg, s, slab], dst_idx)
      for b in range(NBUF):
        pltpu.async_copy(x2.at[src_idx.at[b]], rows.at[b], gsem.at[b])

      @pl.loop(0, NJB, step=NBUF)
      def _chunks(jb):
        for b in range(NBUF):
          j = jb + b
          pltpu.make_async_copy(x2.at[src_idx.at[j]], rows.at[b],
                                gsem.at[b]).wait()
          pltpu.async_copy(rows.at[b], agg_sp.at[dst_idx.at[j]], ssem.at[b],
                           add=True)
          pltpu.async_copy(ones_v, deg_sp.at[dst_idx.at[j]], dsem.at[b],
                           add=True)
          pltpu.make_async_copy(rows.at[b], agg_sp.at[dst_idx.at[j]],
                                ssem.at[b]).wait()
          pltpu.make_async_copy(ones_v, deg_sp.at[dst_idx.at[j]],
                                dsem.at[b]).wait()

          @pl.when(j + NBUF < NJB)
          def _issue():
            pltpu.async_copy(x2.at[src_idx.at[j + NBUF]], rows.at[b],
                             gsem.at[b])

    plsc.subcore_barrier()
    pltpu.sync_copy(agg_sp.at[pl.ds(s * RPT, RPT), :],
                    agg_out.at[sg, pl.ds(s * RPT, RPT), :])
    pltpu.sync_copy(deg_sp.at[pl.ds(s * RPT, RPT)],
                    deg_out.at[sg, 0, pl.ds(s * RPT, RPT)])
    plsc.subcore_barrier()


def _sc_aggregate(x2, srcv, dstv):
  zrow = jnp.zeros((ZR, D), jnp.float32)
  zdeg = jnp.zeros((RPT,), jnp.float32)
  mesh = plsc.VectorSubcoreMesh(core_axis_name="c", subcore_axis_name="s",
                                num_cores=NC, num_subcores=NS)
  f = pl.kernel(
      _sc_body,
      out_type=(jax.ShapeDtypeStruct((RT, NPAD, D), jnp.float32),
                jax.ShapeDtypeStruct((RT, 1, NPAD), jnp.float32)),
      mesh=mesh,
      scratch_types=[
          pltpu.VMEM((NJB, K), jnp.int32),
          pltpu.VMEM((NJB, K), jnp.int32),
          pltpu.VMEM((NBUF, K, D), jnp.float32),
          pltpu.VMEM((K,), jnp.float32),
          pltpu.VMEM((ZR, D), jnp.float32),
          pltpu.VMEM((RPT,), jnp.float32),
          pltpu.VMEM_SHARED((NPAD, D), jnp.float32),
          pltpu.VMEM_SHARED((NPAD,), jnp.float32),
          pltpu.SemaphoreType.DMA((NBUF,)),
          pltpu.SemaphoreType.DMA((NBUF,)),
          pltpu.SemaphoreType.DMA((NBUF,)),
          pltpu.SemaphoreType.DMA,
      ],
  )
  return f(x2, srcv, dstv, zrow, zdeg)


def _elu(x):
  return jnp.where(x > 0, x, jnp.exp(jnp.minimum(x, 0.0)) - 1.0)


def _conv_from_agg(a, d, Wf, bfa, bc):
  dm = jnp.maximum(d, 1.0)
  ind = jnp.minimum(d, 1.0)
  pre = jnp.dot(a / dm, Wf, preferred_element_type=jnp.float32,
                precision=_PREC) + ind * bfa + bc
  return _elu(pre)


def _mask_body(agg, deg, Wa, Wc, ba, bc, wiht, whh, bih, bhh,
               conv_out, mask_out, h_s):
  t = pl.program_id(1)
  Wf = jnp.dot(Wa[...], Wc[...], preferred_element_type=jnp.float32,
               precision=_PREC)
  bfa = jnp.dot(ba[...], Wc[...], preferred_element_type=jnp.float32,
                precision=_PREC)
  conv = _conv_from_agg(agg[0, 0], deg[0, 0], Wf, bfa, bc[...])
  conv_out[0] = conv
  giT = lax.dot_general(wiht[0], conv, (((0,), (1,)), ((), ())),
                        preferred_element_type=jnp.float32,
                        precision=_PREC) + bih[0]

  @pl.when(t == 0)
  def _init():
    h_s[...] = jnp.full((1, NPAD), 0.5, jnp.float32)

  h = h_s[...]
  gh = h * whh[0] + bhh[0]
  rg = jax.nn.sigmoid(giT[0:1] + gh[0:1])
  zg = jax.nn.sigmoid(giT[1:2] + gh[1:2])
  ng = jnp.tanh(giT[2:3] + rg * gh[2:3])
  h = (1.0 - zg) * ng + zg * h
  h_s[...] = h
  real = lax.broadcasted_iota(jnp.int32, (1, NPAD), 1) < N
  val = jnp.sum(jnp.where(real, h, 0.0)) * (1.0 / N)
  sel = lax.broadcasted_iota(jnp.int32, (1, 1, T), 2) == t
  mask_out[...] = jnp.where(sel, val, mask_out[...])


def _tc_masks(aggR, degR, Wa, Wc, ba, bc, wiht, whh, bih, bhh):
  return pl.pallas_call(
      _mask_body,
      grid=(R, T),
      in_specs=[
          pl.BlockSpec((1, 1, NPAD, D), lambda r, t: (r, t, 0, 0)),
          pl.BlockSpec((1, 1, NPAD, 1), lambda r, t: (r, t, 0, 0)),
          pl.BlockSpec((D, D), lambda r, t: (0, 0)),
          pl.BlockSpec((D, D), lambda r, t: (0, 0)),
          pl.BlockSpec((1, D), lambda r, t: (0, 0)),
          pl.BlockSpec((1, D), lambda r, t: (0, 0)),
          pl.BlockSpec((1, D, 3), lambda r, t: (r, 0, 0)),
          pl.BlockSpec((1, 3, 1), lambda r, t: (r, 0, 0)),
          pl.BlockSpec((1, 3, 1), lambda r, t: (r, 0, 0)),
          pl.BlockSpec((1, 3, 1), lambda r, t: (r, 0, 0)),
      ],
      out_specs=[
          pl.BlockSpec((1, NPAD, D), lambda r, t: (r * T + t, 0, 0)),
          pl.BlockSpec((1, 1, T), lambda r, t: (r, 0, 0)),
      ],
      out_shape=[
          jax.ShapeDtypeStruct((RT, NPAD, D), jnp.float32),
          jax.ShapeDtypeStruct((R, 1, T), jnp.float32),
      ],
      scratch_shapes=[pltpu.VMEM((1, NPAD), jnp.float32)],
      compiler_params=pltpu.CompilerParams(
          dimension_semantics=("arbitrary", "arbitrary")),
  )(aggR, degR, Wa, Wc, ba, bc, wiht, whh, bih, bhh)


_BLK = 1024


def _fuse_body(conv, m, gamma, beta, wproj, bproj, out):
  mm = m[:, 0, :]
  ex = jnp.exp(mm - jnp.max(mm, axis=0, keepdims=True))
  w = ex / jnp.sum(ex, axis=0, keepdims=True)
  acc = jnp.zeros((_BLK, D), jnp.float32)
  for t in range(T):
    feat = conv[t] * w[0, t] + conv[T + t] * w[1, t]
    mu = jnp.mean(feat, axis=-1, keepdims=True)
    var = jnp.mean((feat - mu) ** 2, axis=-1, keepdims=True)
    ln = (feat - mu) / jnp.sqrt(var + 1e-5) * gamma[...] + beta[...]
    acc = acc + ln * wproj[0, t]
  out[...] = acc + bproj[0, 0]


def _tc_fuse(conv, m, gamma, beta, wproj, bproj):
  nblk = NPAD // _BLK
  return pl.pallas_call(
      _fuse_body,
      grid=(nblk,),
      in_specs=[
          pl.BlockSpec((RT, _BLK, D), lambda i: (0, i, 0)),
          pl.BlockSpec((R, 1, T), lambda i: (0, 0, 0)),
          pl.BlockSpec((1, D), lambda i: (0, 0)),
          pl.BlockSpec((1, D), lambda i: (0, 0)),
          pl.BlockSpec((1, T), lambda i: (0, 0)),
          pl.BlockSpec((1, 1), lambda i: (0, 0)),
      ],
      out_specs=pl.BlockSpec((_BLK, D), lambda i: (i, 0)),
      out_shape=jax.ShapeDtypeStruct((NPAD, D), jnp.float32),
  )(conv, m, gamma, beta, wproj, bproj)


def kernel(x, llm_feat, W_adapt, b_adapt, W_conv, b_conv, W_ih, W_hh,
           b_ih, b_hh, gamma, beta, W_proj, b_proj, edges):
  del llm_feat
  x2 = x.reshape(T * N, D)
  offs = (jnp.arange(T, dtype=jnp.int32) * N).reshape(1, T, 1)
  srcv = (edges[:, :, 0, :] + offs).reshape(RT, NS, NSLAB, NJB, K)
  dstv = edges[:, :, 1, :].reshape(RT, NS, NSLAB, NJB, K)

  agg, degp = _sc_aggregate(x2, srcv, dstv)
  deg6 = degp.reshape(RT, NPAD)

  aggR = agg.reshape(R, T, NPAD, D)
  degR = deg6.reshape(R, T, NPAD, 1)
  baR = b_adapt.reshape(1, D)
  bcR = b_conv.reshape(1, D)
  wiht = jnp.transpose(W_ih, (0, 2, 1))
  bihR = b_ih.reshape(R, 3, 1)
  bhhR = b_hh.reshape(R, 3, 1)
  conv, masks = _tc_masks(aggR, degR, W_adapt, W_conv, baR, bcR, wiht,
                          W_hh, bihR, bhhR)

  out = _tc_fuse(conv, masks, gamma.reshape(1, D), beta.reshape(1, D),
                 W_proj.reshape(1, T), b_proj.reshape(1, 1))
  return out[:N]


# speedup vs baseline: 1.0785x; 1.0785x over previous
import functools

import jax
import jax.numpy as jnp
from jax import lax
from jax.experimental import pallas as pl
from jax.experimental.pallas import tpu as pltpu
from jax.experimental.pallas import tpu_sc as plsc

N = 10000
E = 320000
R = 2
T = 3
D = 128
RT = R * T

NC = 2
NS = 16
SETS_PER_SC = RT // NC
EPT = E // NS
K = 125
NJC = EPT // K
NJB = 40
NSLAB = NJC // NJB
NBUF = 2
NPAD = 10240
RPT = NPAD // NS
ZR = 16

_PREC = jax.lax.Precision.HIGHEST


def _sc_body(x2, srcv, dstv, zrow, zdeg, agg_out, deg_out,
             src_idx, dst_idx, rows, ones_v, zrow_v, zdeg_v,
             agg_sp, deg_sp, gsem, ssem, dsem, zsem):
  c = lax.axis_index("c")
  s = lax.axis_index("s")
  for i in range(K // 16):
    ones_v[pl.ds(i * 16, 16)] = jnp.ones((16,), jnp.float32)
  pltpu.sync_copy(zrow, zrow_v)
  pltpu.sync_copy(zdeg, zdeg_v)
  for sl in range(SETS_PER_SC):
    sg = c * SETS_PER_SC + sl
    for z in range(RPT // ZR):
      pltpu.async_copy(zrow_v, agg_sp.at[pl.ds(s * RPT + z * ZR, ZR), :],
                       zsem)
    pltpu.sync_copy(zdeg_v, deg_sp.at[pl.ds(s * RPT, RPT)])
    for z in range(RPT // ZR):
      pltpu.make_async_copy(
          zrow_v, agg_sp.at[pl.ds(s * RPT + z * ZR, ZR), :], zsem).wait()
    plsc.subcore_barrier()
    for slab in range(NSLAB):
      pltpu.sync_copy(srcv.at[sg, s, slab], src_idx)
      pltpu.sync_copy(dstv.at[sg, s, slab], dst_idx)
      for b in range(NBUF):
        pltpu.async_copy(x2.at[src_idx.at[b]], rows.at[b], gsem.at[b])

      @pl.loop(0, NJB, step=NBUF)
      def _chunks(jb):
        for b in range(NBUF):
          j = jb + b
          pltpu.make_async_copy(x2.at[src_idx.at[j]], rows.at[b],
                                gsem.at[b]).wait()
          pltpu.async_copy(rows.at[b], agg_sp.at[dst_idx.at[j]], ssem.at[b],
                           add=True)
          pltpu.async_copy(ones_v, deg_sp.at[dst_idx.at[j]], dsem.at[b],
                           add=True)
          pltpu.make_async_copy(rows.at[b], agg_sp.at[dst_idx.at[j]],
                                ssem.at[b]).wait()
          pltpu.make_async_copy(ones_v, deg_sp.at[dst_idx.at[j]],
                                dsem.at[b]).wait()

          @pl.when(j + NBUF < NJB)
          def _issue():
            pltpu.async_copy(x2.at[src_idx.at[j + NBUF]], rows.at[b],
                             gsem.at[b])

    plsc.subcore_barrier()
    pltpu.sync_copy(agg_sp.at[pl.ds(s * RPT, RPT), :],
                    agg_out.at[sg, pl.ds(s * RPT, RPT), :])
    pltpu.sync_copy(deg_sp.at[pl.ds(s * RPT, RPT)],
                    deg_out.at[sg, 0, pl.ds(s * RPT, RPT)])
    plsc.subcore_barrier()


def _sc_aggregate(x2, srcv, dstv):
  zrow = jnp.zeros((ZR, D), jnp.float32)
  zdeg = jnp.zeros((RPT,), jnp.float32)
  mesh = plsc.VectorSubcoreMesh(core_axis_name="c", subcore_axis_name="s",
                                num_cores=NC, num_subcores=NS)
  f = pl.kernel(
      _sc_body,
      out_type=(jax.ShapeDtypeStruct((RT, NPAD, D), jnp.float32),
                jax.ShapeDtypeStruct((RT, 1, NPAD), jnp.float32)),
      mesh=mesh,
      scratch_types=[
          pltpu.VMEM((NJB, K), jnp.int32),
          pltpu.VMEM((NJB, K), jnp.int32),
          pltpu.VMEM((NBUF, K, D), jnp.float32),
          pltpu.VMEM((K,), jnp.float32),
          pltpu.VMEM((ZR, D), jnp.float32),
          pltpu.VMEM((RPT,), jnp.float32),
          pltpu.VMEM_SHARED((NPAD, D), jnp.float32),
          pltpu.VMEM_SHARED((NPAD,), jnp.float32),
          pltpu.SemaphoreType.DMA((NBUF,)),
          pltpu.SemaphoreType.DMA((NBUF,)),
          pltpu.SemaphoreType.DMA((NBUF,)),
          pltpu.SemaphoreType.DMA,
      ],
  )
  return f(x2, srcv, dstv, zrow, zdeg)


def _elu(x):
  return jnp.where(x > 0, x, jnp.exp(jnp.minimum(x, 0.0)) - 1.0)


def _conv_from_agg(a, d, Wf, bfa, bc):
  dm = jnp.maximum(d, 1.0)
  ind = jnp.minimum(d, 1.0)
  pre = jnp.dot(a / dm, Wf, preferred_element_type=jnp.float32,
                precision=_PREC) + ind * bfa + bc
  return _elu(pre)


def _mask_body(agg, deg, Wa, Wc, ba, bc, wiht, whh, bih, bhh,
               conv_out, mask_out, h_s):
  t = pl.program_id(1)
  Wf = jnp.dot(Wa[...], Wc[...], preferred_element_type=jnp.float32,
               precision=_PREC)
  bfa = jnp.dot(ba[...], Wc[...], preferred_element_type=jnp.float32,
                precision=_PREC)
  conv = _conv_from_agg(agg[0, 0], deg[0, 0], Wf, bfa, bc[...])
  conv_out[0] = conv
  giT = lax.dot_general(wiht[0], conv, (((0,), (1,)), ((), ())),
                        preferred_element_type=jnp.float32,
                        precision=_PREC) + bih[0]

  @pl.when(t == 0)
  def _init():
    h_s[...] = jnp.full((1, NPAD), 0.5, jnp.float32)

  h = h_s[...]
  gh = h * whh[0] + bhh[0]
  rg = jax.nn.sigmoid(giT[0:1] + gh[0:1])
  zg = jax.nn.sigmoid(giT[1:2] + gh[1:2])
  ng = jnp.tanh(giT[2:3] + rg * gh[2:3])
  h = (1.0 - zg) * ng + zg * h
  h_s[...] = h
  real = lax.broadcasted_iota(jnp.int32, (1, NPAD), 1) < N
  val = jnp.sum(jnp.where(real, h, 0.0)) * (1.0 / N)
  sel = lax.broadcasted_iota(jnp.int32, (1, 1, T), 2) == t
  mask_out[...] = jnp.where(sel, val, mask_out[...])


def _tc_masks(aggR, degR, Wa, Wc, ba, bc, wiht, whh, bih, bhh):
  return pl.pallas_call(
      _mask_body,
      grid=(R, T),
      in_specs=[
          pl.BlockSpec((1, 1, NPAD, D), lambda r, t: (r, t, 0, 0)),
          pl.BlockSpec((1, 1, NPAD, 1), lambda r, t: (r, t, 0, 0)),
          pl.BlockSpec((D, D), lambda r, t: (0, 0)),
          pl.BlockSpec((D, D), lambda r, t: (0, 0)),
          pl.BlockSpec((1, D), lambda r, t: (0, 0)),
          pl.BlockSpec((1, D), lambda r, t: (0, 0)),
          pl.BlockSpec((1, D, 3), lambda r, t: (r, 0, 0)),
          pl.BlockSpec((1, 3, 1), lambda r, t: (r, 0, 0)),
          pl.BlockSpec((1, 3, 1), lambda r, t: (r, 0, 0)),
          pl.BlockSpec((1, 3, 1), lambda r, t: (r, 0, 0)),
      ],
      out_specs=[
          pl.BlockSpec((1, NPAD, D), lambda r, t: (r * T + t, 0, 0)),
          pl.BlockSpec((1, 1, T), lambda r, t: (r, 0, 0)),
      ],
      out_shape=[
          jax.ShapeDtypeStruct((RT, NPAD, D), jnp.float32),
          jax.ShapeDtypeStruct((R, 1, T), jnp.float32),
      ],
      scratch_shapes=[pltpu.VMEM((1, NPAD), jnp.float32)],
      compiler_params=pltpu.CompilerParams(
          dimension_semantics=("arbitrary", "arbitrary")),
  )(aggR, degR, Wa, Wc, ba, bc, wiht, whh, bih, bhh)


_BLK = 1024


def _fuse_body(conv, m, gamma, beta, wproj, bproj, out):
  mm = m[:, 0, :]
  ex = jnp.exp(mm - jnp.max(mm, axis=0, keepdims=True))
  w = ex / jnp.sum(ex, axis=0, keepdims=True)
  acc = jnp.zeros((_BLK, D), jnp.float32)
  for t in range(T):
    feat = conv[t] * w[0, t] + conv[T + t] * w[1, t]
    mu = jnp.mean(feat, axis=-1, keepdims=True)
    var = jnp.mean((feat - mu) ** 2, axis=-1, keepdims=True)
    ln = (feat - mu) / jnp.sqrt(var + 1e-5) * gamma[...] + beta[...]
    acc = acc + ln * wproj[0, t]
  out[...] = acc + bproj[0, 0]


def _tc_fuse(conv, m, gamma, beta, wproj, bproj):
  nblk = NPAD // _BLK
  return pl.pallas_call(
      _fuse_body,
      grid=(nblk,),
      in_specs=[
          pl.BlockSpec((RT, _BLK, D), lambda i: (0, i, 0)),
          pl.BlockSpec((R, 1, T), lambda i: (0, 0, 0)),
          pl.BlockSpec((1, D), lambda i: (0, 0)),
          pl.BlockSpec((1, D), lambda i: (0, 0)),
          pl.BlockSpec((1, T), lambda i: (0, 0)),
          pl.BlockSpec((1, 1), lambda i: (0, 0)),
      ],
      out_specs=pl.BlockSpec((_BLK, D), lambda i: (i, 0)),
      out_shape=jax.ShapeDtypeStruct((NPAD, D), jnp.float32),
  )(conv, m, gamma, beta, wproj, bproj)


def kernel(x, llm_feat, W_adapt, b_adapt, W_conv, b_conv, W_ih, W_hh,
           b_ih, b_hh, gamma, beta, W_proj, b_proj, edges):
  del llm_feat
  x2 = x.reshape(T * N, D)
  offs = (jnp.arange(T, dtype=jnp.int32) * N).reshape(1, T, 1)
  srcv = (edges[:, :, 0, :] + offs).reshape(RT, NS, NSLAB, NJB, K)
  dstv = edges[:, :, 1, :].reshape(RT, NS, NSLAB, NJB, K)

  agg, degp = _sc_aggregate(x2, srcv, dstv)
  deg6 = degp.reshape(RT, NPAD)

  aggR = agg.reshape(R, T, NPAD, D)
  degR = deg6.reshape(R, T, NPAD, 1)
  baR = b_adapt.reshape(1, D)
  bcR = b_conv.reshape(1, D)
  wiht = jnp.transpose(W_ih, (0, 2, 1))
  bihR = b_ih.reshape(R, 3, 1)
  bhhR = b_hh.reshape(R, 3, 1)
  conv, masks = _tc_masks(aggR, degR, W_adapt, W_conv, baR, bcR, wiht,
                          W_hh, bihR, bhhR)

  out = _tc_fuse(conv, masks, gamma.reshape(1, D), beta.reshape(1, D),
                 W_proj.reshape(1, T), b_proj.reshape(1, 1))
  return out[:N]
